# baseline (speedup 1.0000x reference)
import jax
import jax.numpy as jnp
from jax import lax
from jax.experimental import pallas as pl
from jax.experimental.pallas import tpu as pltpu

_GB = 8


def _gather_body(idx_ref, *refs):
    tok_refs = refs[:_GB]
    pos_ref = refs[_GB]
    out_ref = refs[_GB + 1]
    rows = jnp.concatenate([r[0] for r in tok_refs], axis=0)
    out_ref[...] = (rows + pos_ref[...]).astype(jnp.bfloat16)


def _tc_gather(idx, table, pos):
    (S,) = idx.shape
    V, E = table.shape
    table3 = table.reshape(V, 1, E)
    in_specs = [
        pl.BlockSpec((1, 1, E), (lambda i, idx_ref, k=k: (idx_ref[_GB * i + k], 0, 0)))
        for k in range(_GB)
    ]
    in_specs.append(pl.BlockSpec((_GB, E), lambda i, idx_ref: (i, 0)))
    grid_spec = pltpu.PrefetchScalarGridSpec(
        num_scalar_prefetch=1,
        grid=(S // _GB,),
        in_specs=in_specs,
        out_specs=pl.BlockSpec((_GB, E), lambda i, idx_ref: (i, 0)),
    )
    return pl.pallas_call(
        _gather_body,
        grid_spec=grid_spec,
        out_shape=jax.ShapeDtypeStruct((S, E), jnp.bfloat16),
    )(idx, table3, *([table3] * (_GB - 1)), pos)


def _mm_body(x_ref, w_ref, out_ref):
    out_ref[...] = lax.dot_general(
        x_ref[...],
        w_ref[...].astype(jnp.bfloat16),
        (((1,), (1,)), ((), ())),
        preferred_element_type=jnp.float32,
    )


def kernel(in_idx, tok_emb, pos_emb, W_out):
    B, S = in_idx.shape
    V, E = tok_emb.shape
    x = _tc_gather(in_idx.reshape(-1), tok_emb, pos_emb[:S])

    VB = 512
    logits = pl.pallas_call(
        _mm_body,
        grid=(pl.cdiv(V, VB),),
        in_specs=[
            pl.BlockSpec((S, E), lambda i: (0, 0)),
            pl.BlockSpec((VB, E), lambda i: (i, 0)),
        ],
        out_specs=pl.BlockSpec((S, VB), lambda i: (0, i)),
        out_shape=jax.ShapeDtypeStruct((S, V), jnp.float32),
    )(x, W_out)
    return logits.reshape(B, S, V)


# speedup vs baseline: 1.8554x; 1.8554x over previous
import jax
import jax.numpy as jnp
from jax import lax
from jax.experimental import pallas as pl
from jax.experimental.pallas import tpu as pltpu


def _gather_body(idx_ref, tok_hbm, pos_ref, out_ref, xf_ref, sem):
    S = out_ref.shape[0]

    def issue(i, _):
        pltpu.make_async_copy(
            tok_hbm.at[pl.ds(idx_ref[i], 1), :],
            xf_ref.at[pl.ds(i, 1), :],
            sem,
        ).start()
        return 0

    lax.fori_loop(0, S, issue, 0)

    def drain(i, _):
        pltpu.make_async_copy(
            tok_hbm.at[pl.ds(idx_ref[i], 1), :],
            xf_ref.at[pl.ds(i, 1), :],
            sem,
        ).wait()
        return 0

    lax.fori_loop(0, S, drain, 0)
    out_ref[...] = (xf_ref[...] + pos_ref[...]).astype(jnp.bfloat16)


def _tc_gather(idx, table, pos):
    (S,) = idx.shape
    V, E = table.shape
    grid_spec = pltpu.PrefetchScalarGridSpec(
        num_scalar_prefetch=1,
        grid=(1,),
        in_specs=[
            pl.BlockSpec(memory_space=pltpu.MemorySpace.HBM),
            pl.BlockSpec((S, E), lambda i, idx_ref: (0, 0)),
        ],
        out_specs=pl.BlockSpec((S, E), lambda i, idx_ref: (0, 0)),
        scratch_shapes=[
            pltpu.VMEM((S, E), jnp.float32),
            pltpu.SemaphoreType.DMA,
        ],
    )
    return pl.pallas_call(
        _gather_body,
        grid_spec=grid_spec,
        out_shape=jax.ShapeDtypeStruct((S, E), jnp.bfloat16),
    )(idx, table, pos)


def _mm_body(x_ref, w_ref, out_ref):
    out_ref[...] = lax.dot_general(
        x_ref[...],
        w_ref[...].astype(jnp.bfloat16),
        (((1,), (1,)), ((), ())),
        preferred_element_type=jnp.float32,
    )


def kernel(in_idx, tok_emb, pos_emb, W_out):
    B, S = in_idx.shape
    V, E = tok_emb.shape
    x = _tc_gather(in_idx.reshape(-1), tok_emb, pos_emb[:S])

    VB = 512
    logits = pl.pallas_call(
        _mm_body,
        grid=(pl.cdiv(V, VB),),
        in_specs=[
            pl.BlockSpec((S, E), lambda i: (0, 0)),
            pl.BlockSpec((VB, E), lambda i: (i, 0)),
        ],
        out_specs=pl.BlockSpec((S, VB), lambda i: (0, i)),
        out_shape=jax.ShapeDtypeStruct((S, V), jnp.float32),
    )(x, W_out)
    return logits.reshape(B, S, V)


# speedup vs baseline: 3.6626x; 1.9741x over previous
import functools

import jax
import jax.numpy as jnp
from jax import lax
from jax.experimental import pallas as pl
from jax.experimental.pallas import tpu as pltpu
from jax.experimental.pallas import tpu_sc as plsc

_VB = 512
_N_CHUNKS = 4


def _sc_gather(idx, table):
    (B,) = idx.shape
    V, D = table.shape
    info = plsc.get_sparse_core_info()
    NC, NS = info.num_cores, info.num_subcores
    NW = NC * NS
    b_per_w = B // NW
    mesh = plsc.VectorSubcoreMesh(core_axis_name="c", subcore_axis_name="s")

    @functools.partial(
        pl.kernel,
        mesh=mesh,
        out_type=jax.ShapeDtypeStruct((B, D), jnp.float32),
        scratch_types=[
            pltpu.VMEM((b_per_w,), jnp.int32),
            pltpu.VMEM((b_per_w, D), jnp.float32),
            pltpu.SemaphoreType.DMA,
        ],
    )
    def gather_kernel(idx_hbm, table_hbm, out_hbm, idx_v, rows_v, sem):
        wid = lax.axis_index("s") * NC + lax.axis_index("c")
        base = wid * b_per_w
        pltpu.sync_copy(idx_hbm.at[pl.ds(base, b_per_w)], idx_v)
        pltpu.async_copy(table_hbm.at[idx_v], rows_v, sem).wait()
        pltpu.sync_copy(rows_v, out_hbm.at[pl.ds(base, b_per_w)])

    return gather_kernel(idx, table)


def _mm_body(x_ref, pos_ref, w_ref, out_ref, xs_ref):
    @pl.when(pl.program_id(0) == 0)
    def _():
        xs_ref[...] = (x_ref[...] + pos_ref[...]).astype(jnp.bfloat16)

    out_ref[:, 0, :] = lax.dot_general(
        w_ref[...].astype(jnp.bfloat16),
        xs_ref[...],
        (((1,), (1,)), ((), ())),
        preferred_element_type=jnp.float32,
    )


def _mm_t(x, pos, W_out):
    S, E = x.shape
    V = W_out.shape[0]
    n_tiles = pl.cdiv(V, _VB)
    return pl.pallas_call(
        _mm_body,
        grid=(n_tiles,),
        in_specs=[
            pl.BlockSpec((S, E), lambda i: (0, 0)),
            pl.BlockSpec((S, E), lambda i: (0, 0)),
            pl.BlockSpec((_VB, E), lambda i: (i, 0)),
        ],
        out_specs=pl.BlockSpec((_VB, 1, S), lambda i: (i, 0, 0)),
        out_shape=jax.ShapeDtypeStruct((V, 1, S), jnp.float32),
        scratch_shapes=[pltpu.VMEM((S, E), jnp.bfloat16)],
    )(x, pos, W_out)


def kernel(in_idx, tok_emb, pos_emb, W_out):
    B, S = in_idx.shape
    V, E = tok_emb.shape
    x = _sc_gather(in_idx.reshape(-1), tok_emb)

    logits_t = _mm_t(x, pos_emb[:S], W_out)
    return jnp.transpose(logits_t, (1, 2, 0))


# speedup vs baseline: 4.0982x; 1.1189x over previous
import functools

import jax
import jax.numpy as jnp
from jax import lax
from jax.experimental import pallas as pl
from jax.experimental.pallas import tpu as pltpu
from jax.experimental.pallas import tpu_sc as plsc

_VB = 1024
_N_CHUNKS = 4


def _sc_gather(idx, table):
    (B,) = idx.shape
    V, D = table.shape
    info = plsc.get_sparse_core_info()
    NC, NS = info.num_cores, info.num_subcores
    NW = NC * NS
    b_per_w = B // NW
    mesh = plsc.VectorSubcoreMesh(core_axis_name="c", subcore_axis_name="s")

    @functools.partial(
        pl.kernel,
        mesh=mesh,
        out_type=jax.ShapeDtypeStruct((B, D), jnp.float32),
        scratch_types=[
            pltpu.VMEM((b_per_w,), jnp.int32),
            pltpu.VMEM((b_per_w, D), jnp.float32),
            pltpu.SemaphoreType.DMA,
        ],
    )
    def gather_kernel(idx_hbm, table_hbm, out_hbm, idx_v, rows_v, sem):
        wid = lax.axis_index("s") * NC + lax.axis_index("c")
        base = wid * b_per_w
        pltpu.sync_copy(idx_hbm.at[pl.ds(base, b_per_w)], idx_v)
        pltpu.async_copy(table_hbm.at[idx_v], rows_v, sem).wait()
        pltpu.sync_copy(rows_v, out_hbm.at[pl.ds(base, b_per_w)])

    return gather_kernel(idx, table)


def _mm_body(x_ref, pos_ref, w_ref, out_ref, xs_ref):
    @pl.when(pl.program_id(0) == 0)
    def _():
        xs_ref[...] = (x_ref[...] + pos_ref[...]).astype(jnp.bfloat16)

    out_ref[:, 0, :] = lax.dot_general(
        w_ref[...].astype(jnp.bfloat16),
        xs_ref[...],
        (((1,), (1,)), ((), ())),
        preferred_element_type=jnp.float32,
    )


def _mm_t(x, pos, W_out):
    S, E = x.shape
    V = W_out.shape[0]
    n_tiles = pl.cdiv(V, _VB)
    return pl.pallas_call(
        _mm_body,
        grid=(n_tiles,),
        in_specs=[
            pl.BlockSpec((S, E), lambda i: (0, 0)),
            pl.BlockSpec((S, E), lambda i: (0, 0)),
            pl.BlockSpec((_VB, E), lambda i: (i, 0)),
        ],
        out_specs=pl.BlockSpec((_VB, 1, S), lambda i: (i, 0, 0)),
        out_shape=jax.ShapeDtypeStruct((V, 1, S), jnp.float32),
        scratch_shapes=[pltpu.VMEM((S, E), jnp.bfloat16)],
    )(x, pos, W_out)


def kernel(in_idx, tok_emb, pos_emb, W_out):
    B, S = in_idx.shape
    V, E = tok_emb.shape
    x = _sc_gather(in_idx.reshape(-1), tok_emb)

    logits_t = _mm_t(x, pos_emb[:S], W_out)
    return jnp.transpose(logits_t, (1, 2, 0))
